# baseline (speedup 1.0000x reference)
import functools
import math

import jax
import jax.numpy as jnp
from jax import lax
from jax.experimental import pallas as pl
from jax.experimental.pallas import tpu as pltpu
from jax.experimental.pallas import tpu_sc as plsc

_BATCH = 16384
_NW = 16
_BPW = _BATCH // _NW
_HALF = _BPW // 2
_LANES = 16
_C = math.log(10.0) / 400.0


def _bt_body(idx_a_hbm, idx_b_hbm, elos_hbm, out_hbm,
             ia0_v, ia1_v, ib0_v, ib1_v, ea_v, eb_v, o_v, sem, gsem0, gsem1, osem):
    wid = lax.axis_index("s")
    base = wid * _BPW

    idx_bufs = ((ia0_v, ib0_v), (ia1_v, ib1_v))
    stages = []
    for h in range(2):
        off = base + h * _HALF
        stages.append(pltpu.async_copy(
            idx_a_hbm.at[pl.ds(off, _HALF)], idx_bufs[h][0], sem))
        stages.append(pltpu.async_copy(
            idx_b_hbm.at[pl.ds(off, _HALF)], idx_bufs[h][1], sem))
    for c in stages:
        c.wait()

    gsems = (gsem0, gsem1)
    gathers = []
    for h in range(2):
        gathers.append(pltpu.async_copy(
            elos_hbm.at[idx_bufs[h][0]], ea_v.at[pl.ds(h * _HALF, _HALF)], gsems[h]))
        gathers.append(pltpu.async_copy(
            elos_hbm.at[idx_bufs[h][1]], eb_v.at[pl.ds(h * _HALF, _HALF)], gsems[h]))

    stores = []
    for h in range(2):
        gathers[2 * h].wait()
        gathers[2 * h + 1].wait()
        for k in range(h * (_HALF // _LANES), (h + 1) * (_HALF // _LANES)):
            a = ea_v[pl.ds(k * _LANES, _LANES)]
            b = eb_v[pl.ds(k * _LANES, _LANES)]
            e = jnp.exp((b - a) * _C)
            o_v[pl.ds(k * _LANES, _LANES)] = 1.0 / (1.0 + e)
        stores.append(pltpu.async_copy(
            o_v.at[pl.ds(h * _HALF, _HALF)],
            out_hbm.at[pl.ds(base + h * _HALF, _HALF)], osem))
    for c in stores:
        c.wait()


@jax.jit
def kernel(idx_a, idx_b, elos):
    mesh = plsc.VectorSubcoreMesh(core_axis_name="c", subcore_axis_name="s", num_cores=1)
    run = functools.partial(
        pl.kernel,
        mesh=mesh,
        out_type=jax.ShapeDtypeStruct((_BATCH,), jnp.float32),
        scratch_types=[
            pltpu.VMEM((_HALF,), jnp.int32),
            pltpu.VMEM((_HALF,), jnp.int32),
            pltpu.VMEM((_HALF,), jnp.int32),
            pltpu.VMEM((_HALF,), jnp.int32),
            pltpu.VMEM((_BPW,), jnp.float32),
            pltpu.VMEM((_BPW,), jnp.float32),
            pltpu.VMEM((_BPW,), jnp.float32),
            pltpu.SemaphoreType.DMA,
            pltpu.SemaphoreType.DMA,
            pltpu.SemaphoreType.DMA,
            pltpu.SemaphoreType.DMA,
        ],
    )(_bt_body)
    return run(idx_a.astype(jnp.int32), idx_b.astype(jnp.int32), elos)


# speedup vs baseline: 1.0143x; 1.0143x over previous
import functools
import math

import jax
import jax.numpy as jnp
from jax import lax
from jax.experimental import pallas as pl
from jax.experimental.pallas import tpu as pltpu
from jax.experimental.pallas import tpu_sc as plsc

_BATCH = 16384
_NW = 32
_BPW = _BATCH // _NW
_HALF = _BPW // 2
_LANES = 16
_C = math.log(10.0) / 400.0


def _bt_body(idx_a_hbm, idx_b_hbm, elos_hbm, out_hbm,
             ia0_v, ia1_v, ib0_v, ib1_v, ea_v, eb_v, o_v, sem, ssem1, gsem0, gsem1, osem):
    wid = lax.axis_index("s") * 2 + lax.axis_index("c")
    base = wid * _BPW

    idx_bufs = ((ia0_v, ib0_v), (ia1_v, ib1_v))
    ssems = (sem, ssem1)
    stages = []
    for h in range(2):
        off = base + h * _HALF
        stages.append(pltpu.async_copy(
            idx_a_hbm.at[pl.ds(off, _HALF)], idx_bufs[h][0], ssems[h]))
        stages.append(pltpu.async_copy(
            idx_b_hbm.at[pl.ds(off, _HALF)], idx_bufs[h][1], ssems[h]))

    gsems = (gsem0, gsem1)
    gathers = []
    for h in range(2):
        stages[2 * h].wait()
        stages[2 * h + 1].wait()
        gathers.append(pltpu.async_copy(
            elos_hbm.at[idx_bufs[h][0]], ea_v.at[pl.ds(h * _HALF, _HALF)], gsems[h]))
        gathers.append(pltpu.async_copy(
            elos_hbm.at[idx_bufs[h][1]], eb_v.at[pl.ds(h * _HALF, _HALF)], gsems[h]))

    stores = []
    for h in range(2):
        gathers[2 * h].wait()
        gathers[2 * h + 1].wait()
        for k in range(h * (_HALF // _LANES), (h + 1) * (_HALF // _LANES)):
            a = ea_v[pl.ds(k * _LANES, _LANES)]
            b = eb_v[pl.ds(k * _LANES, _LANES)]
            e = jnp.exp((b - a) * _C)
            o_v[pl.ds(k * _LANES, _LANES)] = 1.0 / (1.0 + e)
        stores.append(pltpu.async_copy(
            o_v.at[pl.ds(h * _HALF, _HALF)],
            out_hbm.at[pl.ds(base + h * _HALF, _HALF)], osem))
    for c in stores:
        c.wait()


@jax.jit
def kernel(idx_a, idx_b, elos):
    mesh = plsc.VectorSubcoreMesh(core_axis_name="c", subcore_axis_name="s")
    run = functools.partial(
        pl.kernel,
        mesh=mesh,
        out_type=jax.ShapeDtypeStruct((_BATCH,), jnp.float32),
        scratch_types=[
            pltpu.VMEM((_HALF,), jnp.int32),
            pltpu.VMEM((_HALF,), jnp.int32),
            pltpu.VMEM((_HALF,), jnp.int32),
            pltpu.VMEM((_HALF,), jnp.int32),
            pltpu.VMEM((_BPW,), jnp.float32),
            pltpu.VMEM((_BPW,), jnp.float32),
            pltpu.VMEM((_BPW,), jnp.float32),
            pltpu.SemaphoreType.DMA,
            pltpu.SemaphoreType.DMA,
            pltpu.SemaphoreType.DMA,
            pltpu.SemaphoreType.DMA,
            pltpu.SemaphoreType.DMA,
        ],
    )(_bt_body)
    return run(idx_a.astype(jnp.int32), idx_b.astype(jnp.int32), elos)
